# baseline (speedup 1.0000x reference)
import functools

import jax
import jax.numpy as jnp
from jax import lax
from jax.experimental import pallas as pl
from jax.experimental.pallas import tpu as pltpu
from jax.experimental.pallas import tpu_sc as plsc

N = 10000
F_IN = 128
H = 64
C = 3
G = 64
E = 320000

NC = 2
NS = 16
NW = NC * NS
LC = 128
CPW = 80
EP = NW * CPW * LC
NP = 10240
RPT = NP // NS
DW = 16
NBUF = 2

@functools.cache
def _sc_kernels():
    mesh = plsc.VectorSubcoreMesh(
        core_axis_name="c", subcore_axis_name="s",
        num_cores=NC, num_subcores=NS)

    @functools.partial(
        pl.kernel,
        out_type=jax.ShapeDtypeStruct((NC, NP, DW), jnp.float32),
        mesh=mesh,
        scratch_types=[
            pltpu.VMEM((CPW, LC), jnp.int32),
            pltpu.VMEM((LC, DW), jnp.float32),
            pltpu.VMEM((LC, DW), jnp.float32),
            pltpu.VMEM_SHARED((NP, DW), jnp.float32),
        ],
    )
    def sc_deg(dst_hbm, out_hbm, dstv, onesv, zerov, acc_sp):
        cid = lax.axis_index("c")
        sid = lax.axis_index("s")
        wid = sid * NC + cid

        def fill(i, carry):
            onesv[i, :] = jnp.ones((DW,), jnp.float32)
            zerov[i, :] = jnp.zeros((DW,), jnp.float32)
            return carry

        lax.fori_loop(0, LC, fill, 0)
        for k in range(RPT // LC):
            pltpu.sync_copy(zerov, acc_sp.at[pl.ds(sid * RPT + k * LC, LC)])
        pltpu.sync_copy(dst_hbm.at[wid], dstv)
        plsc.subcore_barrier()

        def body(j, carry):
            pltpu.sync_copy(onesv, acc_sp.at[dstv.at[j]], add=True)
            return carry

        lax.fori_loop(0, CPW, body, 0)
        plsc.subcore_barrier()
        pltpu.sync_copy(acc_sp.at[pl.ds(sid * RPT, RPT)],
                        out_hbm.at[cid, pl.ds(sid * RPT, RPT)])

    @functools.partial(
        pl.kernel,
        out_type=jax.ShapeDtypeStruct((NC, NP, H), jnp.float32),
        mesh=mesh,
        compiler_params=pltpu.CompilerParams(use_tc_tiling_on_sc=False),
        scratch_types=[
            pltpu.VMEM((CPW, LC), jnp.int32),
            pltpu.VMEM((CPW, LC), jnp.int32),
            tuple(pltpu.VMEM((LC, H), jnp.float32) for _ in range(NBUF)),
            pltpu.VMEM_SHARED((NP, H), jnp.float32),
            pltpu.VMEM_SHARED((NP, H), jnp.float32),
            tuple(pltpu.SemaphoreType.DMA for _ in range(NBUF)),
            tuple(pltpu.SemaphoreType.DMA for _ in range(NBUF)),
        ],
    )
    def sc_agg(hs_hbm, src_hbm, dst_hbm, out_hbm, srcv, dstv, bufs,
               hs_sp, acc_sp, gsem, ssem):
        cid = lax.axis_index("c")
        sid = lax.axis_index("s")
        wid = sid * NC + cid

        def zfill(i, carry):
            for t in range(H // 16):
                bufs[0][i, pl.ds(t * 16, 16)] = jnp.zeros((16,), jnp.float32)
            return carry

        lax.fori_loop(0, LC, zfill, 0)
        for k in range(RPT // LC):
            pltpu.sync_copy(bufs[0], acc_sp.at[pl.ds(sid * RPT + k * LC, LC)])
        pltpu.sync_copy(hs_hbm.at[pl.ds(sid * RPT, RPT)],
                        hs_sp.at[pl.ds(sid * RPT, RPT)])
        pltpu.sync_copy(src_hbm.at[wid], srcv)
        pltpu.sync_copy(dst_hbm.at[wid], dstv)
        plsc.subcore_barrier()

        def body(j, carry):
            pltpu.async_copy(hs_sp.at[srcv.at[j]], bufs[0], gsem[0]).wait()
            pltpu.sync_copy(bufs[0], acc_sp.at[dstv.at[j]], add=True)
            return carry

        lax.fori_loop(0, CPW, body, 0)
        plsc.subcore_barrier()
        pltpu.sync_copy(acc_sp.at[pl.ds(sid * RPT, RPT)],
                        out_hbm.at[cid, pl.ds(sid * RPT, RPT)])

    return sc_deg, sc_agg


def _sc_deg(dst3):
    return _sc_kernels()[0](dst3)


def _sc_agg(hs, src3, dst3):
    return _sc_kernels()[1](hs, src3, dst3)


def _dinv(deg_ref, rows):
    d = deg_ref[0, :rows, 0:1] + deg_ref[1, :rows, 0:1] + 1.0
    return lax.rsqrt(d)


def _tc_pre_body(deg_ref, x_ref, w1_ref, out_ref):
    dinv = _dinv(deg_ref, N)
    h = jnp.dot(x_ref[...], w1_ref[...], preferred_element_type=jnp.float32)
    out_ref[0:N, :] = h * dinv
    out_ref[N:, :] = jnp.zeros((NP - N, H), jnp.float32)


def _tc_mid_body(agg_ref, hs_ref, deg_ref, b1_ref, g1_ref, be1_ref, w2_ref,
                 out_ref):
    dinv = _dinv(deg_ref, N)
    t = (agg_ref[0, 0:N, :] + agg_ref[1, 0:N, :] + hs_ref[0:N, :]) * dinv
    t = t + b1_ref[...]
    mu = jnp.mean(t, axis=0, keepdims=True)
    var = jnp.mean(jnp.square(t - mu), axis=0, keepdims=True)
    hbn = (t - mu) * lax.rsqrt(var + 1e-5) * g1_ref[...] + be1_ref[...]
    hbn = jnp.maximum(hbn, 0.0)
    h2 = jnp.dot(hbn, w2_ref[...], preferred_element_type=jnp.float32)
    out_ref[0:N, :] = h2 * dinv
    out_ref[N:, :] = jnp.zeros((NP - N, H), jnp.float32)


def _tc_final_body(agg_ref, hs_ref, deg_ref, b2_ref, g2_ref, be2_ref,
                   batch_ref, wc_ref, bc_ref, out_ref):
    dinv = _dinv(deg_ref, N)
    t = (agg_ref[0, 0:N, :] + agg_ref[1, 0:N, :] + hs_ref[0:N, :]) * dinv
    t = t + b2_ref[...]
    mu = jnp.mean(t, axis=0, keepdims=True)
    var = jnp.mean(jnp.square(t - mu), axis=0, keepdims=True)
    hbn = (t - mu) * lax.rsqrt(var + 1e-5) * g2_ref[...] + be2_ref[...]
    hbn = jnp.maximum(hbn, 0.0)
    gid = lax.broadcasted_iota(jnp.int32, (G, N), 0)
    p = (batch_ref[...] == gid).astype(jnp.float32)
    psum = jnp.dot(p, hbn, preferred_element_type=jnp.float32)
    cnt = jnp.sum(p, axis=1, keepdims=True)
    pooled = psum / jnp.maximum(cnt, 1.0)
    out_ref[...] = (jnp.dot(pooled, wc_ref[...],
                            preferred_element_type=jnp.float32) + bc_ref[...])


_tc_pre = pl.pallas_call(
    _tc_pre_body, out_shape=jax.ShapeDtypeStruct((NP, H), jnp.float32))
_tc_mid = pl.pallas_call(
    _tc_mid_body, out_shape=jax.ShapeDtypeStruct((NP, H), jnp.float32))
_tc_final = pl.pallas_call(
    _tc_final_body, out_shape=jax.ShapeDtypeStruct((G, 128), jnp.float32))


def kernel(x, edge_index, batch, W1, b1, g1, be1, W2, b2, g2, be2, Wc, bc):
    src = edge_index[0]
    dst = edge_index[1]
    fill = jnp.full((EP - E,), NP - 1, jnp.int32)
    src3 = jnp.concatenate([src, fill]).reshape(NW, CPW, LC)
    dst3 = jnp.concatenate([dst, fill]).reshape(NW, CPW, LC)

    deg2 = _sc_deg(dst3)
    hs1 = _tc_pre(deg2, x, W1)
    agg1 = _sc_agg(hs1, src3, dst3)
    hs2 = _tc_mid(agg1, hs1, deg2, b1.reshape(1, H), g1.reshape(1, H),
                  be1.reshape(1, H), W2)
    agg2 = _sc_agg(hs2, src3, dst3)
    wcp = jnp.pad(Wc, ((0, 0), (0, 128 - C)))
    bcp = jnp.pad(bc, (0, 128 - C)).reshape(1, 128)
    out = _tc_final(agg2, hs2, deg2, b2.reshape(1, H), g2.reshape(1, H),
                    be2.reshape(1, H), batch.reshape(1, N), wcp, bcp)
    return out[:, :C]


# speedup vs baseline: 1.0284x; 1.0284x over previous
import functools

import jax
import jax.numpy as jnp
from jax import lax
from jax.experimental import pallas as pl
from jax.experimental.pallas import tpu as pltpu
from jax.experimental.pallas import tpu_sc as plsc

N = 10000
F_IN = 128
H = 64
C = 3
G = 64
E = 320000

NC = 2
NS = 16
NW = NC * NS
LC = 128
CPW = 80
EP = NW * CPW * LC
NP = 10240
RPT = NP // NS
DW = 16
NBUF = 2

@functools.cache
def _sc_kernels():
    mesh = plsc.VectorSubcoreMesh(
        core_axis_name="c", subcore_axis_name="s",
        num_cores=NC, num_subcores=NS)

    @functools.partial(
        pl.kernel,
        out_type=jax.ShapeDtypeStruct((NC, NP, DW), jnp.float32),
        mesh=mesh,
        scratch_types=[
            pltpu.VMEM((CPW, LC), jnp.int32),
            pltpu.VMEM((LC, DW), jnp.float32),
            pltpu.VMEM((LC, DW), jnp.float32),
            pltpu.VMEM_SHARED((NP, DW), jnp.float32),
        ],
    )
    def sc_deg(dst_hbm, out_hbm, dstv, onesv, zerov, acc_sp):
        cid = lax.axis_index("c")
        sid = lax.axis_index("s")
        wid = sid * NC + cid

        def fill(i, carry):
            onesv[i, :] = jnp.ones((DW,), jnp.float32)
            zerov[i, :] = jnp.zeros((DW,), jnp.float32)
            return carry

        lax.fori_loop(0, LC, fill, 0)
        for k in range(RPT // LC):
            pltpu.sync_copy(zerov, acc_sp.at[pl.ds(sid * RPT + k * LC, LC)])
        pltpu.sync_copy(dst_hbm.at[wid], dstv)
        plsc.subcore_barrier()

        def body(j, carry):
            pltpu.sync_copy(onesv, acc_sp.at[dstv.at[j]], add=True)
            return carry

        lax.fori_loop(0, CPW, body, 0)
        plsc.subcore_barrier()
        pltpu.sync_copy(acc_sp.at[pl.ds(sid * RPT, RPT)],
                        out_hbm.at[cid, pl.ds(sid * RPT, RPT)])

    @functools.partial(
        pl.kernel,
        out_type=jax.ShapeDtypeStruct((NC, NP, H), jnp.float32),
        mesh=mesh,
        compiler_params=pltpu.CompilerParams(use_tc_tiling_on_sc=False),
        scratch_types=[
            pltpu.VMEM((CPW, LC), jnp.int32),
            pltpu.VMEM((CPW, LC), jnp.int32),
            pltpu.VMEM((NBUF, LC, H), jnp.float32),
            pltpu.VMEM_SHARED((NP, H), jnp.float32),
            pltpu.VMEM_SHARED((NP, H), jnp.float32),
            pltpu.SemaphoreType.DMA,
            pltpu.SemaphoreType.DMA,
        ],
    )
    def sc_agg(hs_hbm, src_hbm, dst_hbm, out_hbm, srcv, dstv, buf,
               hs_sp, acc_sp, gsem, ssem):
        cid = lax.axis_index("c")
        sid = lax.axis_index("s")
        wid = sid * NC + cid

        def zfill(i, carry):
            for t in range(H // 16):
                buf[0, i, pl.ds(t * 16, 16)] = jnp.zeros((16,), jnp.float32)
            return carry

        lax.fori_loop(0, LC, zfill, 0)
        for k in range(RPT // LC):
            pltpu.sync_copy(buf.at[0], acc_sp.at[pl.ds(sid * RPT + k * LC, LC)])
        pltpu.sync_copy(hs_hbm.at[pl.ds(sid * RPT, RPT)],
                        hs_sp.at[pl.ds(sid * RPT, RPT)])
        pltpu.sync_copy(src_hbm.at[wid], srcv)
        pltpu.sync_copy(dst_hbm.at[wid], dstv)
        plsc.subcore_barrier()

        def body(g, carry):
            base = g * NBUF
            gd = [pltpu.async_copy(hs_sp.at[srcv.at[base + k]], buf.at[k],
                                   gsem) for k in range(NBUF)]
            for d in gd:
                d.wait()
            sd = [pltpu.async_copy(buf.at[k], acc_sp.at[dstv.at[base + k]],
                                   ssem, add=True) for k in range(NBUF)]
            for d in sd:
                d.wait()
            return carry

        lax.fori_loop(0, CPW // NBUF, body, 0)
        plsc.subcore_barrier()
        pltpu.sync_copy(acc_sp.at[pl.ds(sid * RPT, RPT)],
                        out_hbm.at[cid, pl.ds(sid * RPT, RPT)])

    return sc_deg, sc_agg


def _sc_deg(dst3):
    return _sc_kernels()[0](dst3)


def _sc_agg(hs, src3, dst3):
    return _sc_kernels()[1](hs, src3, dst3)


def _dinv(deg_ref, rows):
    d = deg_ref[0, :rows, 0:1] + deg_ref[1, :rows, 0:1] + 1.0
    return lax.rsqrt(d)


def _tc_pre_body(deg_ref, x_ref, w1_ref, out_ref):
    dinv = _dinv(deg_ref, N)
    h = jnp.dot(x_ref[...], w1_ref[...], preferred_element_type=jnp.float32)
    out_ref[0:N, :] = h * dinv
    out_ref[N:, :] = jnp.zeros((NP - N, H), jnp.float32)


def _tc_mid_body(agg_ref, hs_ref, deg_ref, b1_ref, g1_ref, be1_ref, w2_ref,
                 out_ref):
    dinv = _dinv(deg_ref, N)
    t = (agg_ref[0, 0:N, :] + agg_ref[1, 0:N, :] + hs_ref[0:N, :]) * dinv
    t = t + b1_ref[...]
    mu = jnp.mean(t, axis=0, keepdims=True)
    var = jnp.mean(jnp.square(t - mu), axis=0, keepdims=True)
    hbn = (t - mu) * lax.rsqrt(var + 1e-5) * g1_ref[...] + be1_ref[...]
    hbn = jnp.maximum(hbn, 0.0)
    h2 = jnp.dot(hbn, w2_ref[...], preferred_element_type=jnp.float32)
    out_ref[0:N, :] = h2 * dinv
    out_ref[N:, :] = jnp.zeros((NP - N, H), jnp.float32)


def _tc_final_body(agg_ref, hs_ref, deg_ref, b2_ref, g2_ref, be2_ref,
                   batch_ref, wc_ref, bc_ref, out_ref):
    dinv = _dinv(deg_ref, N)
    t = (agg_ref[0, 0:N, :] + agg_ref[1, 0:N, :] + hs_ref[0:N, :]) * dinv
    t = t + b2_ref[...]
    mu = jnp.mean(t, axis=0, keepdims=True)
    var = jnp.mean(jnp.square(t - mu), axis=0, keepdims=True)
    hbn = (t - mu) * lax.rsqrt(var + 1e-5) * g2_ref[...] + be2_ref[...]
    hbn = jnp.maximum(hbn, 0.0)
    gid = lax.broadcasted_iota(jnp.int32, (G, N), 0)
    p = (batch_ref[...] == gid).astype(jnp.float32)
    psum = jnp.dot(p, hbn, preferred_element_type=jnp.float32)
    cnt = jnp.sum(p, axis=1, keepdims=True)
    pooled = psum / jnp.maximum(cnt, 1.0)
    out_ref[...] = (jnp.dot(pooled, wc_ref[...],
                            preferred_element_type=jnp.float32) + bc_ref[...])


_tc_pre = pl.pallas_call(
    _tc_pre_body, out_shape=jax.ShapeDtypeStruct((NP, H), jnp.float32))
_tc_mid = pl.pallas_call(
    _tc_mid_body, out_shape=jax.ShapeDtypeStruct((NP, H), jnp.float32))
_tc_final = pl.pallas_call(
    _tc_final_body, out_shape=jax.ShapeDtypeStruct((G, 128), jnp.float32))


def kernel(x, edge_index, batch, W1, b1, g1, be1, W2, b2, g2, be2, Wc, bc):
    src = edge_index[0]
    dst = edge_index[1]
    fill = jnp.full((EP - E,), NP - 1, jnp.int32)
    src3 = jnp.concatenate([src, fill]).reshape(NW, CPW, LC)
    dst3 = jnp.concatenate([dst, fill]).reshape(NW, CPW, LC)

    deg2 = _sc_deg(dst3)
    hs1 = _tc_pre(deg2, x, W1)
    agg1 = _sc_agg(hs1, src3, dst3)
    hs2 = _tc_mid(agg1, hs1, deg2, b1.reshape(1, H), g1.reshape(1, H),
                  be1.reshape(1, H), W2)
    agg2 = _sc_agg(hs2, src3, dst3)
    wcp = jnp.pad(Wc, ((0, 0), (0, 128 - C)))
    bcp = jnp.pad(bc, (0, 128 - C)).reshape(1, 128)
    out = _tc_final(agg2, hs2, deg2, b2.reshape(1, H), g2.reshape(1, H),
                    be2.reshape(1, H), batch.reshape(1, N), wcp, bcp)
    return out[:, :C]


# speedup vs baseline: 1.0289x; 1.0005x over previous
import functools

import jax
import jax.numpy as jnp
from jax import lax
from jax.experimental import pallas as pl
from jax.experimental.pallas import tpu as pltpu
from jax.experimental.pallas import tpu_sc as plsc

N = 10000
F_IN = 128
H = 64
C = 3
G = 64
E = 320000

NC = 2
NS = 16
NW = NC * NS
LC = 128
NCH = E // LC
CPW = NCH // NW
XW = NCH - NW * CPW
NP = 10240
RPT = NP // NS
DW = 16
XROW = 80
NBUF = 2

@functools.cache
def _sc_kernels():
    mesh = plsc.VectorSubcoreMesh(
        core_axis_name="c", subcore_axis_name="s",
        num_cores=NC, num_subcores=NS)

    @functools.partial(
        pl.kernel,
        out_type=jax.ShapeDtypeStruct((NC, NP, DW), jnp.float32),
        mesh=mesh,
        compiler_params=pltpu.CompilerParams(use_tc_tiling_on_sc=False),
        scratch_types=[
            pltpu.VMEM((XROW + 1, LC), jnp.int32),
            pltpu.VMEM((LC, DW), jnp.float32),
            pltpu.VMEM((LC, DW), jnp.float32),
            pltpu.VMEM_SHARED((NP, DW), jnp.float32),
        ],
    )
    def sc_deg(dst_hbm, out_hbm, dstv, onesv, zerov, acc_sp):
        cid = lax.axis_index("c")
        sid = lax.axis_index("s")
        wid = sid * NC + cid

        def fill(i, carry):
            onesv[i, :] = jnp.ones((DW,), jnp.float32)
            zerov[i, :] = jnp.zeros((DW,), jnp.float32)
            return carry

        lax.fori_loop(0, LC, fill, 0)
        for k in range(RPT // LC):
            pltpu.sync_copy(zerov, acc_sp.at[pl.ds(sid * RPT + k * LC, LC)])
        pltpu.sync_copy(dst_hbm.at[pl.ds(wid * CPW, CPW)],
                        dstv.at[pl.ds(0, CPW)])

        @pl.when(wid < XW)
        def _():
            pltpu.sync_copy(dst_hbm.at[pl.ds(NW * CPW + wid, 1)],
                            dstv.at[pl.ds(XROW, 1)])

        plsc.subcore_barrier()

        def body(j, carry):
            pltpu.sync_copy(onesv, acc_sp.at[dstv.at[j]], add=True)
            return carry

        lax.fori_loop(0, CPW, body, 0)

        @pl.when(wid < XW)
        def _():
            pltpu.sync_copy(onesv, acc_sp.at[dstv.at[XROW]], add=True)

        plsc.subcore_barrier()
        pltpu.sync_copy(acc_sp.at[pl.ds(sid * RPT, RPT)],
                        out_hbm.at[cid, pl.ds(sid * RPT, RPT)])

    @functools.partial(
        pl.kernel,
        out_type=jax.ShapeDtypeStruct((NC, NP, H), jnp.float32),
        mesh=mesh,
        compiler_params=pltpu.CompilerParams(use_tc_tiling_on_sc=False),
        scratch_types=[
            pltpu.VMEM((XROW + 1, LC), jnp.int32),
            pltpu.VMEM((XROW + 1, LC), jnp.int32),
            pltpu.VMEM((NBUF, LC, H), jnp.float32),
            pltpu.VMEM_SHARED((NP, H), jnp.float32),
            pltpu.VMEM_SHARED((NP, H), jnp.float32),
            pltpu.SemaphoreType.DMA,
            pltpu.SemaphoreType.DMA,
        ],
    )
    def sc_agg(hs_hbm, src_hbm, dst_hbm, out_hbm, srcv, dstv, buf,
               hs_sp, acc_sp, gsem, ssem):
        cid = lax.axis_index("c")
        sid = lax.axis_index("s")
        wid = sid * NC + cid

        def zfill(i, carry):
            for t in range(H // 16):
                buf[0, i, pl.ds(t * 16, 16)] = jnp.zeros((16,), jnp.float32)
            return carry

        lax.fori_loop(0, LC, zfill, 0)
        for k in range(RPT // LC):
            pltpu.sync_copy(buf.at[0], acc_sp.at[pl.ds(sid * RPT + k * LC, LC)])
        pltpu.sync_copy(hs_hbm.at[pl.ds(sid * RPT, RPT)],
                        hs_sp.at[pl.ds(sid * RPT, RPT)])
        pltpu.sync_copy(src_hbm.at[pl.ds(wid * CPW, CPW)],
                        srcv.at[pl.ds(0, CPW)])
        pltpu.sync_copy(dst_hbm.at[pl.ds(wid * CPW, CPW)],
                        dstv.at[pl.ds(0, CPW)])

        @pl.when(wid < XW)
        def _():
            pltpu.sync_copy(src_hbm.at[pl.ds(NW * CPW + wid, 1)],
                            srcv.at[pl.ds(XROW, 1)])
            pltpu.sync_copy(dst_hbm.at[pl.ds(NW * CPW + wid, 1)],
                            dstv.at[pl.ds(XROW, 1)])

        plsc.subcore_barrier()

        def body(g, carry):
            base = g * NBUF
            gd = [pltpu.async_copy(hs_sp.at[srcv.at[base + k]], buf.at[k],
                                   gsem) for k in range(NBUF)]
            for d in gd:
                d.wait()
            sd = [pltpu.async_copy(buf.at[k], acc_sp.at[dstv.at[base + k]],
                                   ssem, add=True) for k in range(NBUF)]
            for d in sd:
                d.wait()
            return carry

        lax.fori_loop(0, CPW // NBUF, body, 0)

        @pl.when(wid < XW)
        def _():
            pltpu.async_copy(hs_sp.at[srcv.at[XROW]], buf.at[0], gsem).wait()
            pltpu.sync_copy(buf.at[0], acc_sp.at[dstv.at[XROW]], add=True)

        plsc.subcore_barrier()
        pltpu.sync_copy(acc_sp.at[pl.ds(sid * RPT, RPT)],
                        out_hbm.at[cid, pl.ds(sid * RPT, RPT)])

    return sc_deg, sc_agg


def _sc_deg(dst3):
    return _sc_kernels()[0](dst3)


def _sc_agg(hs, src3, dst3):
    return _sc_kernels()[1](hs, src3, dst3)


def _dinv(deg_ref, rows):
    d = deg_ref[0, :rows, 0:1] + deg_ref[1, :rows, 0:1] + 1.0
    return lax.rsqrt(d)


def _tc_mm_body(x_ref, w1_ref, out_ref):
    out_ref[...] = jnp.dot(x_ref[...], w1_ref[...],
                           preferred_element_type=jnp.float32)


def _tc_scale_body(deg_ref, h_ref, out_ref):
    dinv = _dinv(deg_ref, N)
    out_ref[0:N, :] = h_ref[...] * dinv
    out_ref[N:, :] = jnp.zeros((NP - N, H), jnp.float32)


def _tc_mid_body(agg_ref, hs_ref, deg_ref, b1_ref, g1_ref, be1_ref, w2_ref,
                 out_ref):
    dinv = _dinv(deg_ref, N)
    t = (agg_ref[0, 0:N, :] + agg_ref[1, 0:N, :] + hs_ref[0:N, :]) * dinv
    t = t + b1_ref[...]
    mu = jnp.mean(t, axis=0, keepdims=True)
    var = jnp.mean(jnp.square(t - mu), axis=0, keepdims=True)
    hbn = (t - mu) * lax.rsqrt(var + 1e-5) * g1_ref[...] + be1_ref[...]
    hbn = jnp.maximum(hbn, 0.0)
    h2 = jnp.dot(hbn, w2_ref[...], preferred_element_type=jnp.float32)
    out_ref[0:N, :] = h2 * dinv
    out_ref[N:, :] = jnp.zeros((NP - N, H), jnp.float32)


def _tc_final_body(agg_ref, hs_ref, deg_ref, b2_ref, g2_ref, be2_ref,
                   batch_ref, wc_ref, bc_ref, out_ref):
    dinv = _dinv(deg_ref, N)
    t = (agg_ref[0, 0:N, :] + agg_ref[1, 0:N, :] + hs_ref[0:N, :]) * dinv
    t = t + b2_ref[...]
    mu = jnp.mean(t, axis=0, keepdims=True)
    var = jnp.mean(jnp.square(t - mu), axis=0, keepdims=True)
    hbn = (t - mu) * lax.rsqrt(var + 1e-5) * g2_ref[...] + be2_ref[...]
    hbn = jnp.maximum(hbn, 0.0)
    gid = lax.broadcasted_iota(jnp.int32, (G, N), 0)
    p = (batch_ref[...] == gid).astype(jnp.float32)
    psum = jnp.dot(p, hbn, preferred_element_type=jnp.float32)
    cnt = jnp.sum(p, axis=1, keepdims=True)
    pooled = psum / jnp.maximum(cnt, 1.0)
    out_ref[...] = (jnp.dot(pooled, wc_ref[...],
                            preferred_element_type=jnp.float32) + bc_ref[...])


_tc_mm = pl.pallas_call(
    _tc_mm_body, out_shape=jax.ShapeDtypeStruct((N, H), jnp.float32))
_tc_scale = pl.pallas_call(
    _tc_scale_body, out_shape=jax.ShapeDtypeStruct((NP, H), jnp.float32))
_tc_mid = pl.pallas_call(
    _tc_mid_body, out_shape=jax.ShapeDtypeStruct((NP, H), jnp.float32))
_tc_final = pl.pallas_call(
    _tc_final_body, out_shape=jax.ShapeDtypeStruct((G, 128), jnp.float32))


def kernel(x, edge_index, batch, W1, b1, g1, be1, W2, b2, g2, be2, Wc, bc):
    src2 = edge_index[0].reshape(NCH, LC)
    dst2 = edge_index[1].reshape(NCH, LC)

    deg2 = _sc_deg(dst2)
    h1 = _tc_mm(x, W1)
    hs1 = _tc_scale(deg2, h1)
    agg1 = _sc_agg(hs1, src2, dst2)
    hs2 = _tc_mid(agg1, hs1, deg2, b1.reshape(1, H), g1.reshape(1, H),
                  be1.reshape(1, H), W2)
    agg2 = _sc_agg(hs2, src2, dst2)
    wcp = jnp.pad(Wc, ((0, 0), (0, 128 - C)))
    bcp = jnp.pad(bc, (0, 128 - C)).reshape(1, 128)
    out = _tc_final(agg2, hs2, deg2, b2.reshape(1, H), g2.reshape(1, H),
                    be2.reshape(1, H), batch.reshape(1, N), wcp, bcp)
    return out[:, :C]


# speedup vs baseline: 1.0695x; 1.0395x over previous
import functools

import jax
import jax.numpy as jnp
from jax import lax
from jax.experimental import pallas as pl
from jax.experimental.pallas import tpu as pltpu
from jax.experimental.pallas import tpu_sc as plsc

N = 10000
F_IN = 128
H = 64
C = 3
G = 64
E = 320000

NC = 2
NS = 16
NW = NC * NS
LC = 128
NCH = E // LC
CPW = NCH // NW
XW = NCH - NW * CPW
NP = 10240
RPT = NP // NS
DW = 16
XROW = 80
NBUF = 2

@functools.cache
def _sc_kernels():
    mesh = plsc.VectorSubcoreMesh(
        core_axis_name="c", subcore_axis_name="s",
        num_cores=NC, num_subcores=NS)

    @functools.partial(
        pl.kernel,
        out_type=jax.ShapeDtypeStruct((NC, NP, DW), jnp.float32),
        mesh=mesh,
        compiler_params=pltpu.CompilerParams(use_tc_tiling_on_sc=False),
        scratch_types=[
            pltpu.VMEM((XROW + 1, LC), jnp.int32),
            pltpu.VMEM((LC, DW), jnp.float32),
            pltpu.VMEM((LC, DW), jnp.float32),
            pltpu.VMEM_SHARED((NP, DW), jnp.float32),
        ],
    )
    def sc_deg(e_hbm, out_hbm, dstv, onesv, zerov, acc_sp):
        cid = lax.axis_index("c")
        sid = lax.axis_index("s")
        wid = sid * NC + cid

        def fill(i, carry):
            onesv[i, :] = jnp.ones((DW,), jnp.float32)
            zerov[i, :] = jnp.zeros((DW,), jnp.float32)
            return carry

        lax.fori_loop(0, LC, fill, 0)
        for k in range(RPT // LC):
            pltpu.sync_copy(zerov, acc_sp.at[pl.ds(sid * RPT + k * LC, LC)])
        pltpu.sync_copy(e_hbm.at[1, pl.ds(wid * CPW, CPW)],
                        dstv.at[pl.ds(0, CPW)])

        @pl.when(wid < XW)
        def _():
            pltpu.sync_copy(e_hbm.at[1, pl.ds(NW * CPW + wid, 1)],
                            dstv.at[pl.ds(XROW, 1)])

        plsc.subcore_barrier()

        def body(j, carry):
            pltpu.sync_copy(onesv, acc_sp.at[dstv.at[j]], add=True)
            return carry

        lax.fori_loop(0, CPW, body, 0)

        @pl.when(wid < XW)
        def _():
            pltpu.sync_copy(onesv, acc_sp.at[dstv.at[XROW]], add=True)

        plsc.subcore_barrier()
        pltpu.sync_copy(acc_sp.at[pl.ds(sid * RPT, RPT)],
                        out_hbm.at[cid, pl.ds(sid * RPT, RPT)])

    @functools.partial(
        pl.kernel,
        out_type=jax.ShapeDtypeStruct((NC, NP, H), jnp.float32),
        mesh=mesh,
        compiler_params=pltpu.CompilerParams(use_tc_tiling_on_sc=False),
        scratch_types=[
            pltpu.VMEM((XROW + 1, LC), jnp.int32),
            pltpu.VMEM((XROW + 1, LC), jnp.int32),
            pltpu.VMEM((NBUF, LC, H), jnp.float32),
            pltpu.VMEM_SHARED((NP, H), jnp.float32),
            pltpu.VMEM_SHARED((NP, H), jnp.float32),
            pltpu.SemaphoreType.DMA,
            pltpu.SemaphoreType.DMA,
        ],
    )
    def sc_agg(hs_hbm, e_hbm, out_hbm, srcv, dstv, buf,
               hs_sp, acc_sp, gsem, ssem):
        cid = lax.axis_index("c")
        sid = lax.axis_index("s")
        wid = sid * NC + cid

        def zfill(i, carry):
            for t in range(H // 16):
                buf[0, i, pl.ds(t * 16, 16)] = jnp.zeros((16,), jnp.float32)
            return carry

        lax.fori_loop(0, LC, zfill, 0)
        for k in range(RPT // LC):
            pltpu.sync_copy(buf.at[0], acc_sp.at[pl.ds(sid * RPT + k * LC, LC)])
        pltpu.sync_copy(hs_hbm.at[pl.ds(sid * RPT, RPT)],
                        hs_sp.at[pl.ds(sid * RPT, RPT)])
        pltpu.sync_copy(e_hbm.at[0, pl.ds(wid * CPW, CPW)],
                        srcv.at[pl.ds(0, CPW)])
        pltpu.sync_copy(e_hbm.at[1, pl.ds(wid * CPW, CPW)],
                        dstv.at[pl.ds(0, CPW)])

        @pl.when(wid < XW)
        def _():
            pltpu.sync_copy(e_hbm.at[0, pl.ds(NW * CPW + wid, 1)],
                            srcv.at[pl.ds(XROW, 1)])
            pltpu.sync_copy(e_hbm.at[1, pl.ds(NW * CPW + wid, 1)],
                            dstv.at[pl.ds(XROW, 1)])

        plsc.subcore_barrier()

        def body(g, carry):
            base = g * NBUF
            gd = [pltpu.async_copy(hs_sp.at[srcv.at[base + k]], buf.at[k],
                                   gsem) for k in range(NBUF)]
            for d in gd:
                d.wait()
            sd = [pltpu.async_copy(buf.at[k], acc_sp.at[dstv.at[base + k]],
                                   ssem, add=True) for k in range(NBUF)]
            for d in sd:
                d.wait()
            return carry

        lax.fori_loop(0, CPW // NBUF, body, 0)

        @pl.when(wid < XW)
        def _():
            pltpu.async_copy(hs_sp.at[srcv.at[XROW]], buf.at[0], gsem).wait()
            pltpu.sync_copy(buf.at[0], acc_sp.at[dstv.at[XROW]], add=True)

        plsc.subcore_barrier()
        pltpu.sync_copy(acc_sp.at[pl.ds(sid * RPT, RPT)],
                        out_hbm.at[cid, pl.ds(sid * RPT, RPT)])

    return sc_deg, sc_agg


def _sc_deg(e3):
    return _sc_kernels()[0](e3)


def _sc_agg(hs, e3):
    return _sc_kernels()[1](hs, e3)


def _dinv(deg_ref, rows):
    d = deg_ref[0, :rows, 0:1] + deg_ref[1, :rows, 0:1] + 1.0
    return lax.rsqrt(d)


def _tc_mm_body(x_ref, w1_ref, out_ref):
    out_ref[...] = jnp.dot(x_ref[...], w1_ref[...],
                           preferred_element_type=jnp.float32)


def _tc_scale_body(deg_ref, h_ref, out_ref):
    dinv = _dinv(deg_ref, N)
    out_ref[0:N, :] = h_ref[...] * dinv
    out_ref[N:, :] = jnp.zeros((NP - N, H), jnp.float32)


def _tc_mid_body(agg_ref, hs_ref, deg_ref, b1_ref, g1_ref, be1_ref, w2_ref,
                 out_ref):
    dinv = _dinv(deg_ref, N)
    t = (agg_ref[0, 0:N, :] + agg_ref[1, 0:N, :] + hs_ref[0:N, :]) * dinv
    t = t + b1_ref[...]
    mu = jnp.mean(t, axis=0, keepdims=True)
    var = jnp.mean(jnp.square(t - mu), axis=0, keepdims=True)
    hbn = (t - mu) * lax.rsqrt(var + 1e-5) * g1_ref[...] + be1_ref[...]
    hbn = jnp.maximum(hbn, 0.0)
    h2 = jnp.dot(hbn, w2_ref[...], preferred_element_type=jnp.float32)
    out_ref[0:N, :] = h2 * dinv
    out_ref[N:, :] = jnp.zeros((NP - N, H), jnp.float32)


def _tc_final_body(agg_ref, hs_ref, deg_ref, b2_ref, g2_ref, be2_ref,
                   batch_ref, wc_ref, bc_ref, out_ref):
    dinv = _dinv(deg_ref, N)
    t = (agg_ref[0, 0:N, :] + agg_ref[1, 0:N, :] + hs_ref[0:N, :]) * dinv
    t = t + b2_ref[...]
    mu = jnp.mean(t, axis=0, keepdims=True)
    var = jnp.mean(jnp.square(t - mu), axis=0, keepdims=True)
    hbn = (t - mu) * lax.rsqrt(var + 1e-5) * g2_ref[...] + be2_ref[...]
    hbn = jnp.maximum(hbn, 0.0)
    gid = lax.broadcasted_iota(jnp.int32, (G, N), 0)
    p = (batch_ref[...] == gid).astype(jnp.float32)
    psum = jnp.dot(p, hbn, preferred_element_type=jnp.float32)
    cnt = jnp.sum(p, axis=1, keepdims=True)
    pooled = psum / jnp.maximum(cnt, 1.0)
    out_ref[...] = (jnp.dot(pooled, wc_ref[...],
                            preferred_element_type=jnp.float32) + bc_ref[...])


_tc_mm = pl.pallas_call(
    _tc_mm_body, out_shape=jax.ShapeDtypeStruct((N, H), jnp.float32))
_tc_scale = pl.pallas_call(
    _tc_scale_body, out_shape=jax.ShapeDtypeStruct((NP, H), jnp.float32))
_tc_mid = pl.pallas_call(
    _tc_mid_body, out_shape=jax.ShapeDtypeStruct((NP, H), jnp.float32))
_tc_final = pl.pallas_call(
    _tc_final_body, out_shape=jax.ShapeDtypeStruct((G, 128), jnp.float32))


def kernel(x, edge_index, batch, W1, b1, g1, be1, W2, b2, g2, be2, Wc, bc):
    e3 = edge_index.reshape(2, NCH, LC)

    deg2 = _sc_deg(e3)
    h1 = _tc_mm(x, W1)
    hs1 = _tc_scale(deg2, h1)
    agg1 = _sc_agg(hs1, e3)
    hs2 = _tc_mid(agg1, hs1, deg2, b1.reshape(1, H), g1.reshape(1, H),
                  be1.reshape(1, H), W2)
    agg2 = _sc_agg(hs2, e3)
    wcp = jnp.pad(Wc, ((0, 0), (0, 128 - C)))
    bcp = jnp.pad(bc, (0, 128 - C)).reshape(1, 128)
    out = _tc_final(agg2, hs2, deg2, b2.reshape(1, H), g2.reshape(1, H),
                    be2.reshape(1, H), batch.reshape(1, N), wcp, bcp)
    return out[:, :C]
